# baseline (speedup 1.0000x reference)
import functools

import jax
import jax.numpy as jnp
from jax import lax
from jax.experimental import pallas as pl
from jax.experimental.pallas import tpu as pltpu
from jax.experimental.pallas import tpu_sc as plsc

NUM_FREQS = 8
TABLE_ROWS = 256
_INFO = plsc.get_sparse_core_info()
NC, NS, LANES = _INFO.num_cores, _INFO.num_subcores, _INFO.num_lanes
NW = NC * NS
ROWS_PER_CHUNK = 32


def _make_sc_lookup(B: int, L: int):
    feat = L * NUM_FREQS
    rows_per_w = B // NW
    n_chunks = rows_per_w // ROWS_PER_CHUNK
    assert rows_per_w % ROWS_PER_CHUNK == 0 and n_chunks % 2 == 0
    full_groups = L // LANES
    tail = L % LANES
    mesh = plsc.VectorSubcoreMesh(core_axis_name="c", subcore_axis_name="s")

    @functools.partial(
        pl.kernel,
        mesh=mesh,
        out_type=jax.ShapeDtypeStruct((B, feat), jnp.float32),
        scratch_types=[
            pltpu.VMEM((TABLE_ROWS * NUM_FREQS,), jnp.float32),
            pltpu.VMEM((ROWS_PER_CHUNK, L), jnp.int32),
            pltpu.VMEM((ROWS_PER_CHUNK, L), jnp.int32),
            pltpu.VMEM((ROWS_PER_CHUNK, feat), jnp.float32),
            pltpu.VMEM((ROWS_PER_CHUNK, feat), jnp.float32),
            pltpu.SemaphoreType.DMA,
            pltpu.SemaphoreType.DMA,
            pltpu.SemaphoreType.DMA,
            pltpu.SemaphoreType.DMA,
        ],
        compiler_params=pltpu.CompilerParams(
            use_tc_tiling_on_sc=True, needs_layout_passes=False),
    )
    def sc_lookup(idx_hbm, table_hbm, out_hbm,
                  table_v, idx_v0, idx_v1, rows_v0, rows_v1,
                  si0, si1, ss0, ss1):
        wid = lax.axis_index("s") * NC + lax.axis_index("c")
        base = wid * rows_per_w
        idx_v, rows_v = (idx_v0, idx_v1), (rows_v0, rows_v1)
        si, ss = (si0, si1), (ss0, ss1)

        def idx_load(i, b):
            return pltpu.make_async_copy(
                idx_hbm.at[pl.ds(base + i * ROWS_PER_CHUNK, ROWS_PER_CHUNK), :],
                idx_v[b], si[b])

        def store(i, b):
            return pltpu.make_async_copy(
                rows_v[b],
                out_hbm.at[pl.ds(base + i * ROWS_PER_CHUNK, ROWS_PER_CHUNK), :],
                ss[b])

        pltpu.sync_copy(table_hbm, table_v)
        idx_load(0, 0).start()

        lanes = lax.iota(jnp.int32, LANES)
        offs07 = lanes & (NUM_FREQS - 1)
        pat_base = lanes >> 3
        pats = [pat_base + (2 * g) for g in range(LANES // 2)]
        dnums = lax.GatherDimensionNumbers(
            offset_dims=(), collapsed_slice_dims=(0,), start_index_map=(0,))

        def compute(b):
            iv, rv = idx_v[b], rows_v[b]

            def do_group(rr, l0, p0):
                idx8 = iv[rr, pl.ds(l0, LANES)] << 3
                for g in range(LANES // 2):
                    rep = lax.gather(idx8, pats[g][:, None], dnums, (1,),
                                     mode=lax.GatherScatterMode.PROMISE_IN_BOUNDS)
                    r = plsc.load_gather(table_v, [rep + offs07])
                    rv[rr, pl.ds(p0 + g * LANES, LANES)] = r

            @plsc.parallel_loop(0, ROWS_PER_CHUNK, unroll=2)
            def per_row(rr):
                for q in range(full_groups):
                    do_group(rr, q * LANES, q * LANES * NUM_FREQS)
                if tail:
                    do_group(rr, L - LANES, (L - LANES) * NUM_FREQS)

        def body(g, carry):
            for b in (0, 1):
                i = 2 * g + b

                @pl.when(i >= 2)
                def _():
                    store(i - 2, b).wait()

                idx_load(i, b).wait()

                @pl.when(i + 1 < n_chunks)
                def _():
                    idx_load(i + 1, 1 - b).start()

                compute(b)
                store(i, b).start()
            return carry

        lax.fori_loop(0, n_chunks // 2, body, 0)
        store(n_chunks - 2, 0).wait()
        store(n_chunks - 1, 1).wait()

    return sc_lookup


def kernel(tensor, extra_embeddings, int_to_feat_matrix):
    B, L = tensor.shape
    table = jnp.concatenate(
        [int_to_feat_matrix, extra_embeddings.astype(jnp.float32)], axis=0
    )
    table_flat = table.reshape(TABLE_ROWS * NUM_FREQS)
    return _make_sc_lookup(B, L)(tensor, table_flat)


# speedup vs baseline: 1.0969x; 1.0969x over previous
import functools

import jax
import jax.numpy as jnp
from jax import lax
from jax.experimental import pallas as pl
from jax.experimental.pallas import tpu as pltpu
from jax.experimental.pallas import tpu_sc as plsc

NUM_FREQS = 8
TABLE_ROWS = 256
_INFO = plsc.get_sparse_core_info()
NC, NS, LANES = _INFO.num_cores, _INFO.num_subcores, _INFO.num_lanes
NW = NC * NS
ROWS_PER_CHUNK = 32


def _make_sc_lookup(B: int, L: int):
    feat = L * NUM_FREQS
    rows_per_w = B // NW
    n_chunks = rows_per_w // ROWS_PER_CHUNK
    assert rows_per_w % ROWS_PER_CHUNK == 0 and n_chunks % 2 == 0
    full_groups = L // LANES
    tail = L % LANES
    mesh = plsc.VectorSubcoreMesh(core_axis_name="c", subcore_axis_name="s")

    @functools.partial(
        pl.kernel,
        mesh=mesh,
        out_type=jax.ShapeDtypeStruct((B, feat), jnp.float32),
        scratch_types=[
            pltpu.VMEM((TABLE_ROWS * NUM_FREQS,), jnp.float32),
            pltpu.VMEM((ROWS_PER_CHUNK, L), jnp.int32),
            pltpu.VMEM((ROWS_PER_CHUNK, L), jnp.int32),
            pltpu.VMEM((ROWS_PER_CHUNK, feat), jnp.float32),
            pltpu.VMEM((ROWS_PER_CHUNK, feat), jnp.float32),
            pltpu.SemaphoreType.DMA,
            pltpu.SemaphoreType.DMA,
            pltpu.SemaphoreType.DMA,
            pltpu.SemaphoreType.DMA,
        ],
        compiler_params=pltpu.CompilerParams(
            use_tc_tiling_on_sc=True, needs_layout_passes=False),
    )
    def sc_lookup(idx_hbm, table_hbm, out_hbm,
                  table_v, idx_v0, idx_v1, rows_v0, rows_v1,
                  si0, si1, ss0, ss1):
        wid = lax.axis_index("s") * NC + lax.axis_index("c")
        base = wid * rows_per_w
        idx_v, rows_v = (idx_v0, idx_v1), (rows_v0, rows_v1)
        si, ss = (si0, si1), (ss0, ss1)

        def idx_load(i, b):
            return pltpu.make_async_copy(
                idx_hbm.at[pl.ds(base + i * ROWS_PER_CHUNK, ROWS_PER_CHUNK), :],
                idx_v[b], si[b])

        def store(i, b):
            return pltpu.make_async_copy(
                rows_v[b],
                out_hbm.at[pl.ds(base + i * ROWS_PER_CHUNK, ROWS_PER_CHUNK), :],
                ss[b])

        pltpu.sync_copy(table_hbm, table_v)
        idx_load(0, 0).start()

        lanes = lax.iota(jnp.int32, LANES)
        offs07 = lanes & (NUM_FREQS - 1)
        pat_base = lanes >> 3
        pats = [pat_base + (2 * g) for g in range(LANES // 2)]
        dnums = lax.GatherDimensionNumbers(
            offset_dims=(), collapsed_slice_dims=(0,), start_index_map=(0,))

        def compute(b):
            iv, rv = idx_v[b], rows_v[b]

            def do_group(rr, l0, p0):
                idx8 = iv[rr, pl.ds(l0, LANES)] << 3
                for g in range(LANES // 2):
                    rep = lax.gather(idx8, pats[g][:, None], dnums, (1,),
                                     mode=lax.GatherScatterMode.PROMISE_IN_BOUNDS)
                    r = plsc.load_gather(table_v, [rep + offs07])
                    rv[rr, pl.ds(p0 + g * LANES, LANES)] = r

            @plsc.parallel_loop(0, ROWS_PER_CHUNK, unroll=2)
            def per_row(rr):
                for q in range(full_groups):
                    do_group(rr, q * LANES, q * LANES * NUM_FREQS)
                if tail:
                    do_group(rr, L - LANES, (L - LANES) * NUM_FREQS)

        def body(g, carry):
            for b in (0, 1):
                i = 2 * g + b

                @pl.when(i >= 2)
                def _():
                    store(i - 2, b).wait()

                idx_load(i, b).wait()

                @pl.when(i + 1 < n_chunks)
                def _():
                    idx_load(i + 1, 1 - b).start()

                compute(b)
                plsc.subcore_barrier()
                store(i, b).start()
            return carry

        lax.fori_loop(0, n_chunks // 2, body, 0)
        store(n_chunks - 2, 0).wait()
        store(n_chunks - 1, 1).wait()

    return sc_lookup


def kernel(tensor, extra_embeddings, int_to_feat_matrix):
    B, L = tensor.shape
    table = jnp.concatenate(
        [int_to_feat_matrix, extra_embeddings.astype(jnp.float32)], axis=0
    )
    table_flat = table.reshape(TABLE_ROWS * NUM_FREQS)
    return _make_sc_lookup(B, L)(tensor, table_flat)
